# baseline (speedup 1.0000x reference)
import functools
import jax
import jax.numpy as jnp
from jax import lax
from jax.experimental import pallas as pl
from jax.experimental.pallas import tpu as pltpu
from jax.experimental.pallas import tpu_sc as plsc

_RES = 1024
_NBINS = _RES * _RES
_MIS_X = 0.001
_MIS_Y = -0.002

_NC = 2
_NS = 16
_NW = _NC * _NS
_NPART = 2000000
_PER_W = 62496
_CHUNK = 2016
_NCHUNK = _PER_W // _CHUNK
_VPC = _CHUNK // 16
_SLAB = _NBINS // _NS
_DUMP = _NBINS
_PBLK = 16384


def _edges():
    return jnp.linspace(-_RES * 0.001 / 2, _RES * 0.001 / 2, _RES + 1)


def _prep_body(a_ref, px_ref, py_ref):
    def rnd(x):
        u = lax.bitcast_convert_type(x, jnp.int32)
        u = u + ((u >> 16) & 1) + jnp.int32(0x7FFF)
        u = u & jnp.int32(-65536)
        return lax.bitcast_convert_type(u, jnp.float32)

    a = a_ref[...]
    px_ref[...] = rnd(a[:, 0]) - jnp.float32(_MIS_X)
    py_ref[...] = rnd(a[:, 2]) - jnp.float32(_MIS_Y)


def _prep(particles):
    return pl.pallas_call(
        _prep_body,
        grid=((_NPART + _PBLK - 1) // _PBLK,),
        in_specs=[pl.BlockSpec((_PBLK, 7), lambda i: (i, 0))],
        out_specs=[pl.BlockSpec((_PBLK,), lambda i: (i,)),
                   pl.BlockSpec((_PBLK,), lambda i: (i,))],
        out_shape=[jax.ShapeDtypeStruct((_NPART,), jnp.float32),
                   jax.ShapeDtypeStruct((_NPART,), jnp.float32)],
    )(particles)


def _sc_hist(px, py, edges_pad):
    mesh = plsc.VectorSubcoreMesh(core_axis_name="c", subcore_axis_name="s",
                                  num_cores=_NC, num_subcores=_NS)

    @functools.partial(
        pl.kernel,
        out_type=jax.ShapeDtypeStruct((_NC * _NBINS,), jnp.float32),
        mesh=mesh,
        scratch_types=[
            pltpu.VMEM((_CHUNK,), jnp.float32),
            pltpu.VMEM((_CHUNK,), jnp.float32),
            pltpu.VMEM((_CHUNK,), jnp.float32),
            pltpu.VMEM((_CHUNK,), jnp.float32),
            pltpu.VMEM((1032,), jnp.float32),
            pltpu.VMEM((16, 128), jnp.int32),
            pltpu.VMEM((16, 128), jnp.int32),
            pltpu.VMEM((128,), jnp.float32),
            pltpu.VMEM((4096,), jnp.float32),
            pltpu.VMEM_SHARED((_NBINS + 8,), jnp.float32),
            pltpu.SemaphoreType.DMA,
            pltpu.SemaphoreType.DMA,
            pltpu.SemaphoreType.DMA,
            pltpu.SemaphoreType.DMA,
            pltpu.SemaphoreType.DMA,
        ],
        compiler_params=pltpu.CompilerParams(needs_layout_passes=False),
    )
    def hist_kernel(px_hbm, py_hbm, edges_hbm, out_hbm,
                    xbuf0, xbuf1, ybuf0, ybuf1, ebuf, ibuf0, ibuf1,
                    ones, zbuf, hist_s, sem_x0, sem_x1, sem_y0, sem_y1,
                    sem_sc):
        cid = lax.axis_index("c")
        sid = lax.axis_index("s")
        wid = cid * _NS + sid

        iota = lax.iota(jnp.int32, 16)
        zeros_f = jnp.zeros((16,), jnp.float32)
        ones_f = jnp.ones((16,), jnp.float32)
        dump_v = jnp.full((16,), _DUMP, jnp.int32)

        pltpu.sync_copy(edges_hbm, ebuf)

        def oset(v, _):
            ones[pl.ds(v * 16, 16)] = ones_f
            ibuf0[15, pl.ds(96 + v * 16, 16)] = dump_v
            ibuf1[15, pl.ds(96 + v * 16, 16)] = dump_v
            return _
        lax.fori_loop(0, 2, oset, None)

        def oset2(v, _):
            ones[pl.ds(32 + v * 16, 16)] = ones_f
            return _
        lax.fori_loop(0, 6, oset2, None)

        def zset(v, _):
            zbuf[pl.ds(v * 16, 16)] = zeros_f
            return _
        lax.fori_loop(0, 256, zset, None)

        def zcpy(i, _):
            pltpu.sync_copy(zbuf, hist_s.at[pl.ds(sid * _SLAB + i * 4096, 4096)])
            return _
        lax.fori_loop(0, _SLAB // 4096, zcpy, None)

        plsc.subcore_barrier()

        e_lo = jnp.full((16,), jnp.float32(-_RES * 0.001 / 2))
        e_hi = jnp.full((16,), jnp.float32(_RES * 0.001 / 2))

        def binify(p):
            inr = (p >= e_lo) & (p <= e_hi)
            t = jnp.clip((p - e_lo) * jnp.float32(1000.0),
                         jnp.float32(0.0), jnp.float32(1023.0))
            j0 = t.astype(jnp.int32)
            ej = plsc.load_gather(ebuf, [j0])
            ej1 = plsc.load_gather(ebuf, [j0 + 1])
            j = j0 + jnp.where(p >= ej1, 1, 0) - jnp.where(p < ej, 1, 0)
            j = jnp.clip(j, 0, _RES - 1)
            return j, inr

        def compute_vreg(xbuf, ybuf, ibuf, v):
            jx, inx = binify(xbuf[pl.ds(v * 16, 16)])
            jy, iny = binify(ybuf[pl.ds(v * 16, 16)])
            flat = (jnp.int32(_RES - 1) - jy) * _RES + jx
            flat = jnp.where(inx & iny, flat, dump_v)
            ibuf[v >> 3, pl.ds((v & 7) * 16, 16)] = flat

        def in_copies(c, xbuf, ybuf, sx, sy):
            src = wid * _PER_W + c * _CHUNK
            return (pltpu.make_async_copy(px_hbm.at[pl.ds(src, _CHUNK)], xbuf, sx),
                    pltpu.make_async_copy(py_hbm.at[pl.ds(src, _CHUNK)], ybuf, sy))

        for cc, (xb, yb, sx, sy) in ((0, (xbuf0, ybuf0, sem_x0, sem_y0)),
                                     (1, (xbuf1, ybuf1, sem_x1, sem_y1))):
            dx, dy = in_copies(cc, xb, yb, sx, sy)
            dx.start()
            dy.start()

        def process(c, xbuf, ybuf, ibuf, sx, sy):
            dx, dy = in_copies(c, xbuf, ybuf, sx, sy)
            dx.wait()
            dy.wait()

            def vbody(v, _):
                compute_vreg(xbuf, ybuf, ibuf, v)
                return _
            lax.fori_loop(0, _VPC, vbody, None)

            @pl.when(c + 2 < _NCHUNK)
            def _():
                nx, ny = in_copies(c + 2, xbuf, ybuf, sx, sy)
                nx.start()
                ny.start()

            def sbody(r, _):
                pltpu.async_copy(ones, hist_s.at[ibuf.at[r]], sem_sc, add=True)
                return _
            lax.fori_loop(0, 16, sbody, None)

            def dbody(r, _):
                pltpu.make_async_copy(ones, hist_s.at[ibuf.at[r]], sem_sc).wait()
                return _
            lax.fori_loop(0, 16, dbody, None)

        def chunk_pair(g, _):
            c = g * 2
            process(c, xbuf0, ybuf0, ibuf0, sem_x0, sem_y0)

            @pl.when(c + 1 < _NCHUNK)
            def _():
                process(c + 1, xbuf1, ybuf1, ibuf1, sem_x1, sem_y1)
            return _
        lax.fori_loop(0, (_NCHUNK + 1) // 2, chunk_pair, None)

        @pl.when(wid < 8)
        def _tail():
            src = _PER_W * _NW + wid * 16
            pltpu.sync_copy(px_hbm.at[pl.ds(src, 16)], xbuf0.at[pl.ds(0, 16)])
            pltpu.sync_copy(py_hbm.at[pl.ds(src, 16)], ybuf0.at[pl.ds(0, 16)])
            compute_vreg(xbuf0, ybuf0, ibuf0, 0)

            def clr(v, _):
                ibuf0[0, pl.ds(v * 16, 16)] = dump_v
                return _
            lax.fori_loop(1, 8, clr, None)
            pltpu.sync_copy(ones, hist_s.at[ibuf0.at[0]], add=True)

        plsc.subcore_barrier()

        pltpu.sync_copy(hist_s.at[pl.ds(sid * _SLAB, _SLAB)],
                        out_hbm.at[pl.ds(cid * _NBINS + sid * _SLAB, _SLAB)])

    return hist_kernel(px, py, edges_pad)


def _merge_body(p_ref, o_ref):
    o_ref[...] = p_ref[0] + p_ref[1]


def _merge(partials):
    return pl.pallas_call(
        _merge_body,
        grid=(8,),
        in_specs=[pl.BlockSpec((2, 128, _RES), lambda i: (0, i, 0))],
        out_specs=pl.BlockSpec((128, _RES), lambda i: (i, 0)),
        out_shape=jax.ShapeDtypeStruct((_RES, _RES), jnp.float32),
    )(partials)


@jax.jit
def kernel(particles, energy):
    del energy
    edges = _edges().astype(jnp.float32)
    edges_pad = jnp.concatenate([edges, jnp.zeros((7,), jnp.float32)])
    px, py = _prep(particles)
    partials = _sc_hist(px, py, edges_pad)
    return _merge(partials.reshape(_NC, _RES, _RES))


# speedup vs baseline: 2.2335x; 2.2335x over previous
import functools
import jax
import jax.numpy as jnp
from jax import lax
from jax.experimental import pallas as pl
from jax.experimental.pallas import tpu as pltpu
from jax.experimental.pallas import tpu_sc as plsc

_RES = 1024
_NBINS = _RES * _RES
_MIS_X = 0.001
_MIS_Y = -0.002

_NC = 2
_NS = 16
_NW = _NC * _NS
_NPART = 2000000
_PER_W = 62496
_CHUNK = 2016
_NCHUNK = _PER_W // _CHUNK
_VPC = _CHUNK // 16
_SLAB = _NBINS // _NS
_DUMP = _NBINS


def _edges():
    return jnp.linspace(-_RES * 0.001 / 2, _RES * 0.001 / 2, _RES + 1)


def _sc_hist(parts_flat, edges_pad):
    mesh = plsc.VectorSubcoreMesh(core_axis_name="c", subcore_axis_name="s",
                                  num_cores=_NC, num_subcores=_NS)

    @functools.partial(
        pl.kernel,
        out_type=jax.ShapeDtypeStruct((_NC * _NBINS,), jnp.float32),
        mesh=mesh,
        scratch_types=[
            pltpu.VMEM((_CHUNK * 7,), jnp.float32),
            pltpu.VMEM((_CHUNK * 7,), jnp.float32),
            pltpu.VMEM((1032,), jnp.float32),
            pltpu.VMEM((16, 128), jnp.int32),
            pltpu.VMEM((16, 128), jnp.int32),
            pltpu.VMEM((128,), jnp.float32),
            pltpu.VMEM((4096,), jnp.float32),
            pltpu.VMEM_SHARED((_NBINS + 8,), jnp.float32),
            pltpu.SemaphoreType.DMA,
            pltpu.SemaphoreType.DMA,
            pltpu.SemaphoreType.DMA,
            pltpu.SemaphoreType.DMA,
        ],
        compiler_params=pltpu.CompilerParams(needs_layout_passes=False),
    )
    def hist_kernel(parts_hbm, edges_hbm, out_hbm,
                    pbuf0, pbuf1, ebuf, ibuf0, ibuf1, ones, zbuf, hist_s,
                    sem_in0, sem_in1, sem_sc0, sem_sc1):
        cid = lax.axis_index("c")
        sid = lax.axis_index("s")
        wid = cid * _NS + sid

        iota = lax.iota(jnp.int32, 16)
        iota7 = iota * 7
        zeros_f = jnp.zeros((16,), jnp.float32)
        ones_f = jnp.ones((16,), jnp.float32)
        dump_v = jnp.full((16,), _DUMP, jnp.int32)

        pltpu.sync_copy(edges_hbm, ebuf)

        def oset(v, _):
            ones[pl.ds(v * 16, 16)] = ones_f
            ibuf0[15, pl.ds(96 + v * 16, 16)] = dump_v
            ibuf1[15, pl.ds(96 + v * 16, 16)] = dump_v
            return _
        lax.fori_loop(0, 2, oset, None)

        def oset2(v, _):
            ones[pl.ds(32 + v * 16, 16)] = ones_f
            return _
        lax.fori_loop(0, 6, oset2, None)

        def zset(v, _):
            zbuf[pl.ds(v * 16, 16)] = zeros_f
            return _
        lax.fori_loop(0, 256, zset, None)

        def zcpy(i, _):
            pltpu.sync_copy(zbuf, hist_s.at[pl.ds(sid * _SLAB + i * 4096, 4096)])
            return _
        lax.fori_loop(0, _SLAB // 4096, zcpy, None)

        plsc.subcore_barrier()

        e_lo = jnp.full((16,), jnp.float32(-_RES * 0.001 / 2))
        e_hi = jnp.full((16,), jnp.float32(_RES * 0.001 / 2))

        def bf16_round(vals):
            u = plsc.bitcast(vals, jnp.int32)
            u = u + ((u >> 16) & 1) + jnp.int32(0x7FFF)
            u = u & jnp.int32(-65536)
            return plsc.bitcast(u, jnp.float32)

        def binify(vals, mis):
            p = bf16_round(vals) - jnp.float32(mis)
            inr = (p >= e_lo) & (p <= e_hi)
            t = jnp.clip((p - e_lo) * jnp.float32(1000.0),
                         jnp.float32(0.0), jnp.float32(1023.0))
            j0 = t.astype(jnp.int32)
            ej = plsc.load_gather(ebuf, [j0])
            ej1 = plsc.load_gather(ebuf, [j0 + 1])
            j = j0 + jnp.where(p >= ej1, 1, 0) - jnp.where(p < ej, 1, 0)
            j = jnp.clip(j, 0, _RES - 1)
            return j, inr

        def compute_vreg(pbuf, ibuf, v):
            base7 = v * 112
            xg = plsc.load_gather(pbuf, [base7 + iota7])
            yg = plsc.load_gather(pbuf, [base7 + iota7 + 2])
            jx, inx = binify(xg, _MIS_X)
            jy, iny = binify(yg, _MIS_Y)
            flat = (jnp.int32(_RES - 1) - jy) * _RES + jx
            flat = jnp.where(inx & iny, flat, dump_v)
            ibuf[v >> 3, pl.ds((v & 7) * 16, 16)] = flat

        def in_copy(c, pbuf, sem):
            src = (wid * _PER_W + c * _CHUNK) * 7
            return pltpu.make_async_copy(
                parts_hbm.at[pl.ds(src, _CHUNK * 7)], pbuf, sem)

        in_copy(0, pbuf0, sem_in0).start()
        in_copy(1, pbuf1, sem_in1).start()

        def process(c, pbuf, ibuf, sem_in, sem_sc):
            in_copy(c, pbuf, sem_in).wait()

            def vbody(v, _):
                compute_vreg(pbuf, ibuf, v)
                return _
            lax.fori_loop(0, _VPC, vbody, None)

            @pl.when(c + 2 < _NCHUNK)
            def _():
                in_copy(c + 2, pbuf, sem_in).start()

            def sbody(r, _):
                pltpu.async_copy(ones, hist_s.at[ibuf.at[r]], sem_sc, add=True)
                return _
            lax.fori_loop(0, 16, sbody, None)

        def drain(ibuf, sem_sc):
            def dbody(r, _):
                pltpu.make_async_copy(ones, hist_s.at[ibuf.at[r]], sem_sc).wait()
                return _
            lax.fori_loop(0, 16, dbody, None)

        def chunk_pair(g, _):
            c = g * 2

            @pl.when(g > 0)
            def _():
                drain(ibuf0, sem_sc0)
            process(c, pbuf0, ibuf0, sem_in0, sem_sc0)

            @pl.when(c + 1 < _NCHUNK)
            def _():
                @pl.when(g > 0)
                def _():
                    drain(ibuf1, sem_sc1)
                process(c + 1, pbuf1, ibuf1, sem_in1, sem_sc1)
            return _
        lax.fori_loop(0, (_NCHUNK + 1) // 2, chunk_pair, None)

        drain(ibuf0, sem_sc0)
        drain(ibuf1, sem_sc1)

        @pl.when(wid < 8)
        def _tail():
            src = (_PER_W * _NW + wid * 16) * 7
            pltpu.sync_copy(parts_hbm.at[pl.ds(src, 112)], pbuf0.at[pl.ds(0, 112)])
            compute_vreg(pbuf0, ibuf0, 0)

            def clr(v, _):
                ibuf0[0, pl.ds(v * 16, 16)] = dump_v
                return _
            lax.fori_loop(1, 8, clr, None)
            pltpu.sync_copy(ones, hist_s.at[ibuf0.at[0]], add=True)

        plsc.subcore_barrier()

        pltpu.sync_copy(hist_s.at[pl.ds(sid * _SLAB, _SLAB)],
                        out_hbm.at[pl.ds(cid * _NBINS + sid * _SLAB, _SLAB)])

    return hist_kernel(parts_flat, edges_pad)


def _merge_body(p_ref, o_ref):
    o_ref[...] = p_ref[0] + p_ref[1]


def _merge(partials):
    return pl.pallas_call(
        _merge_body,
        grid=(8,),
        in_specs=[pl.BlockSpec((2, 128, _RES), lambda i: (0, i, 0))],
        out_specs=pl.BlockSpec((128, _RES), lambda i: (i, 0)),
        out_shape=jax.ShapeDtypeStruct((_RES, _RES), jnp.float32),
    )(partials)


@jax.jit
def kernel(particles, energy):
    del energy
    edges = _edges().astype(jnp.float32)
    edges_pad = jnp.concatenate([edges, jnp.zeros((7,), jnp.float32)])
    parts_flat = particles.reshape(-1)
    partials = _sc_hist(parts_flat, edges_pad)
    return _merge(partials.reshape(_NC, _RES, _RES))
